# Initial kernel scaffold; baseline (speedup 1.0000x reference)
import jax
import jax.numpy as jnp
from jax.experimental import pallas as pl


def kernel(x, w1, b1, w2, b2):
    raise NotImplementedError("write your pallas kernel here")


# speedup vs baseline: 1.4648x; 1.4648x over previous
import functools

import jax
import jax.numpy as jnp
from jax import lax
from jax.experimental import pallas as pl
from jax.experimental.pallas import tpu as pltpu


def _cam_fused_kernel(x_ref, w1_ref, b1_ref, w2_ref, b2_ref, o_ref, *, hw):
    x = x_ref[0]
    c = x.shape[0]

    if hw >= 128 and hw % 128 == 0:
        acc_s = x[:, 0:128]
        acc_m = x[:, 0:128]
        for j in range(1, hw // 128):
            sl = x[:, j * 128:(j + 1) * 128]
            acc_s = acc_s + sl
            acc_m = jnp.maximum(acc_m, sl)
    else:
        acc_s = x
        acc_m = x
    mean_col = jnp.sum(acc_s, axis=1, keepdims=True) * (1.0 / float(hw))
    max_col = jnp.max(acc_m, axis=1, keepdims=True)

    col = lax.broadcasted_iota(jnp.int32, (c, 2), 1)
    v = jnp.where(col == 0, mean_col, max_col)
    h = jnp.dot(w1_ref[...], v, preferred_element_type=jnp.float32) + b1_ref[...]
    h = jnp.maximum(h, 0.0)
    o = jnp.dot(w2_ref[...], h, preferred_element_type=jnp.float32) + b2_ref[...]
    o = jax.nn.sigmoid(o)
    gate = jax.nn.sigmoid(jnp.sum(o, axis=1, keepdims=True))

    o_ref[0] = x * gate


def kernel(x, w1, b1, w2, b2):
    N, C, H, W = x.shape
    HW = H * W
    hidden = w1.shape[0]

    x_flat = x.reshape(N, C, HW)
    b1c = b1.reshape(hidden, 1)
    b2c = b2.reshape(C, 1)

    cost = pl.CostEstimate(
        flops=3 * N * C * HW + 8 * N * C * hidden,
        transcendentals=3 * N * C,
        bytes_accessed=2 * N * C * HW * 4
        + int(w1.size + b1.size + w2.size + b2.size) * 4,
    )
    out = pl.pallas_call(
        functools.partial(_cam_fused_kernel, hw=HW),
        out_shape=jax.ShapeDtypeStruct((N, C, HW), x.dtype),
        grid=(N,),
        in_specs=[
            pl.BlockSpec((1, C, HW), lambda n: (n, 0, 0)),
            pl.BlockSpec((hidden, C), lambda n: (0, 0)),
            pl.BlockSpec((hidden, 1), lambda n: (0, 0)),
            pl.BlockSpec((C, hidden), lambda n: (0, 0)),
            pl.BlockSpec((C, 1), lambda n: (0, 0)),
        ],
        out_specs=pl.BlockSpec((1, C, HW), lambda n: (n, 0, 0)),
        compiler_params=pltpu.CompilerParams(
            dimension_semantics=("parallel",),
            vmem_limit_bytes=64 * 1024 * 1024),
        cost_estimate=cost,
    )(x_flat, w1, b1c, w2, b2c)

    return out.reshape(N, C, H, W)


# speedup vs baseline: 1.5655x; 1.0688x over previous
import functools

import jax
import jax.numpy as jnp
from jax import lax
from jax.experimental import pallas as pl
from jax.experimental.pallas import tpu as pltpu


def _cam_fused_kernel(x_ref, w1_ref, b1_ref, w2_ref, b2_ref, o_ref, *, hw):
    nb = x_ref.shape[0]
    for b in range(nb):
        x = x_ref[b]
        c = x.shape[0]

        if hw >= 128 and hw % 128 == 0:
            acc_s = x[:, 0:128]
            acc_m = x[:, 0:128]
            for j in range(1, hw // 128):
                sl = x[:, j * 128:(j + 1) * 128]
                acc_s = acc_s + sl
                acc_m = jnp.maximum(acc_m, sl)
        else:
            acc_s = x
            acc_m = x
        mean_col = jnp.sum(acc_s, axis=1, keepdims=True) * (1.0 / float(hw))
        max_col = jnp.max(acc_m, axis=1, keepdims=True)

        col = lax.broadcasted_iota(jnp.int32, (c, 2), 1)
        v = jnp.where(col == 0, mean_col, max_col)
        h = jnp.dot(w1_ref[...], v, preferred_element_type=jnp.float32) + b1_ref[...]
        h = jnp.maximum(h, 0.0)
        o = jnp.dot(w2_ref[...], h, preferred_element_type=jnp.float32) + b2_ref[...]
        o = jax.nn.sigmoid(o)
        gate = jax.nn.sigmoid(jnp.sum(o, axis=1, keepdims=True))

        o_ref[b] = x * gate


def kernel(x, w1, b1, w2, b2):
    N, C, H, W = x.shape
    HW = H * W
    hidden = w1.shape[0]

    x_flat = x.reshape(N, C, HW)
    b1c = b1.reshape(hidden, 1)
    b2c = b2.reshape(C, 1)

    cost = pl.CostEstimate(
        flops=3 * N * C * HW + 8 * N * C * hidden,
        transcendentals=3 * N * C,
        bytes_accessed=2 * N * C * HW * 4
        + int(w1.size + b1.size + w2.size + b2.size) * 4,
    )
    nb = 1
    for cand in (4, 2, 1):
        if N % cand == 0 and cand * C * HW * 4 <= 8 * 1024 * 1024:
            nb = cand
            break
    out = pl.pallas_call(
        functools.partial(_cam_fused_kernel, hw=HW),
        out_shape=jax.ShapeDtypeStruct((N, C, HW), x.dtype),
        grid=(N // nb,),
        in_specs=[
            pl.BlockSpec((nb, C, HW), lambda n: (n, 0, 0)),
            pl.BlockSpec((hidden, C), lambda n: (0, 0)),
            pl.BlockSpec((hidden, 1), lambda n: (0, 0)),
            pl.BlockSpec((C, hidden), lambda n: (0, 0)),
            pl.BlockSpec((C, 1), lambda n: (0, 0)),
        ],
        out_specs=pl.BlockSpec((nb, C, HW), lambda n: (n, 0, 0)),
        compiler_params=pltpu.CompilerParams(
            dimension_semantics=("parallel",),
            vmem_limit_bytes=64 * 1024 * 1024),
        cost_estimate=cost,
    )(x_flat, w1, b1c, w2, b2c)

    return out.reshape(N, C, H, W)
